# baseline (device time: 139176 ns/iter reference)
import os

import jax
import jax.numpy as jnp
from jax import lax
from jax.experimental import pallas as pl
from jax.experimental.pallas import tpu as pltpu

_KPHASES = int(os.environ.get("KPHASES", "3"))

N_RING = 8
BLK = 1024
HALF = BLK // 2
QRT = 128
LPD = HALF // QRT
M_HALF = 1024
NSLOT = 4


def _ring_pos(y, z):
    return jnp.where(y == 0, z, N_RING - 1 - z)


def _ring_to_yz(r):
    y = jnp.where(r >= N_RING // 2, 1, 0)
    z = jnp.where(r < N_RING // 2, r, N_RING - 1 - r)
    return y, z


def kernel(x, dy):
    k, m = x.shape
    _, f = dy.shape

    my_y = lax.axis_index("y")
    my_z = lax.axis_index("z")
    r = _ring_pos(my_y, my_z)
    dy_blk = lax.dynamic_slice(dy, (0, r * BLK), (k, BLK)).astype(jnp.bfloat16)
    xb = x.astype(jnp.bfloat16)

    def body(x_ref, dy_ref, out_ref, bsend, brecv, comm_cw, comm_ccw, stage,
             out_sems, xsend_sems, xrecv_sems,
             cw_send_sems, cw_recv_sems, ccw_send_sems, ccw_recv_sems):
        my_x = lax.axis_index("x")
        my_y = lax.axis_index("y")
        my_z = lax.axis_index("z")
        r = _ring_pos(my_y, my_z)
        ry, rz = _ring_to_yz((r + 1) % N_RING)
        ly, lz = _ring_to_yz((r - 1) % N_RING)
        right = (my_x, ry, rz)
        left = (my_x, ly, lz)

        if _KPHASES >= 2:
            barrier = pltpu.get_barrier_semaphore()
            for tgt in [(1 - my_x, my_y, my_z), right, left]:
                pl.semaphore_signal(barrier, inc=1, device_id=tgt,
                                    device_id_type=pl.DeviceIdType.MESH)
            pl.semaphore_wait(barrier, 3)

        dyb = dy_ref[:, :]

        lanes = []
        for c in range(LPD):
            lanes.append(dict(cw=True, c=c, col0=c * QRT))
            lanes.append(dict(cw=False, c=c, col0=HALF + c * QRT))
        for i, ln in enumerate(lanes):
            ln["xidx"] = i
            ln["comm"] = comm_cw if ln["cw"] else comm_ccw
            ln["ss"] = cw_send_sems if ln["cw"] else ccw_send_sems
            ln["rs"] = cw_recv_sems if ln["cw"] else ccw_recv_sems
            ln["tgt"] = right if ln["cw"] else left

        other0 = (1 - my_x) * M_HALF
        b = lax.dot_general(x_ref[:, pl.ds(other0, M_HALF)], dyb,
                            (((0,), (0,)), ((), ())),
                            preferred_element_type=jnp.float32)
        for ln in lanes:
            bsend[ln["xidx"], :, :] = (
                b[:, ln["col0"]:ln["col0"] + QRT].astype(jnp.bfloat16))

        xchg = []
        if _KPHASES >= 2:
            for ln in lanes:
                i = ln["xidx"]
                d = pltpu.make_async_remote_copy(
                    src_ref=bsend.at[i], dst_ref=brecv.at[i],
                    send_sem=xsend_sems.at[i], recv_sem=xrecv_sems.at[i],
                    device_id=(1 - my_x, my_y, my_z),
                    device_id_type=pl.DeviceIdType.MESH,
                )
                d.start()
                xchg.append(d)

        mine0 = my_x * M_HALF
        a = lax.dot_general(x_ref[:, pl.ds(mine0, M_HALF)], dyb,
                            (((0,), (0,)), ((), ())),
                            preferred_element_type=jnp.float32)

        def ring_desc(ln, h):
            s = h % NSLOT
            rv = (h + 1) % NSLOT
            return pltpu.make_async_remote_copy(
                src_ref=ln["comm"].at[ln["c"], s],
                dst_ref=ln["comm"].at[ln["c"], rv],
                send_sem=ln["ss"].at[ln["c"], s],
                recv_sem=ln["rs"].at[ln["c"], rv],
                device_id=ln["tgt"], device_id_type=pl.DeviceIdType.MESH,
            )

        def out_copy(ln, slot, origin):
            i = ln["xidx"]
            stage[i, :, :] = (
                ln["comm"][ln["c"], slot, :, :].astype(jnp.float32))
            return pltpu.make_async_copy(
                stage.at[i],
                out_ref.at[:, pl.ds(origin * BLK + ln["col0"], QRT)],
                out_sems.at[i])

        for ln in lanes:
            i = ln["xidx"]
            if _KPHASES >= 2:
                xchg[i].wait()
                own_c = (a[:, ln["col0"]:ln["col0"] + QRT]
                         + brecv[i, :, :].astype(jnp.float32))
            else:
                own_c = (a[:, ln["col0"]:ln["col0"] + QRT]
                         + b[:, ln["col0"]:ln["col0"] + QRT])
            ln["comm"][ln["c"], 0, :, :] = own_c.astype(jnp.bfloat16)
            if _KPHASES >= 3:
                d = ring_desc(ln, 0)
                d.start()
                ln["prev"] = d
            i = ln["xidx"]
            stage[i, :, :] = own_c
            ln["prev_out"] = pltpu.make_async_copy(
                stage.at[i],
                out_ref.at[:, pl.ds(r * BLK + ln["col0"], QRT)],
                out_sems.at[i])
            ln["prev_out"].start()

        if _KPHASES < 3:
            for ln in lanes:
                ln["prev_out"].wait()
            return

        for h in range(1, N_RING - 1):
            s = h % NSLOT
            for ln in lanes:
                ln["prev"].wait_recv()
                d = ring_desc(ln, h)
                if h >= NSLOT:
                    d.wait_send()
                d.start()
                ln["prev"] = d
                o = (r - h) % N_RING if ln["cw"] else (r + h) % N_RING
                ln["prev_out"].wait()
                ln["prev_out"] = out_copy(ln, s, o)
                ln["prev_out"].start()

        s = (N_RING - 1) % NSLOT
        for ln in lanes:
            ln["prev"].wait_recv()
            o = (r + 1) % N_RING if ln["cw"] else (r - 1) % N_RING
            ln["prev_out"].wait()
            ln["prev_out"] = out_copy(ln, s, o)
            ln["prev_out"].start()

        for h in range(max(0, N_RING - 1 - NSLOT), N_RING - 1):
            for ln in lanes:
                ring_desc(ln, h).wait_send()

        for ln in lanes:
            ln["prev_out"].wait()

    return pl.pallas_call(
        body,
        out_shape=jax.ShapeDtypeStruct((M_HALF, f), jnp.float32),
        in_specs=[
            pl.BlockSpec(memory_space=pltpu.VMEM),
            pl.BlockSpec(memory_space=pltpu.VMEM),
        ],
        out_specs=pl.BlockSpec(memory_space=pl.ANY),
        scratch_shapes=[
            pltpu.VMEM((2 * LPD, M_HALF, QRT), jnp.bfloat16),
            pltpu.VMEM((2 * LPD, M_HALF, QRT), jnp.bfloat16),
            pltpu.VMEM((LPD, NSLOT, M_HALF, QRT), jnp.bfloat16),
            pltpu.VMEM((LPD, NSLOT, M_HALF, QRT), jnp.bfloat16),
            pltpu.VMEM((2 * LPD, M_HALF, QRT), jnp.float32),
            pltpu.SemaphoreType.DMA((2 * LPD,)),
            pltpu.SemaphoreType.DMA((2 * LPD,)),
            pltpu.SemaphoreType.DMA((2 * LPD,)),
            pltpu.SemaphoreType.DMA((LPD, NSLOT)),
            pltpu.SemaphoreType.DMA((LPD, NSLOT)),
            pltpu.SemaphoreType.DMA((LPD, NSLOT)),
            pltpu.SemaphoreType.DMA((LPD, NSLOT)),
        ],
        compiler_params=(pltpu.CompilerParams(collective_id=0)
                         if _KPHASES >= 2 else pltpu.CompilerParams()),
    )(xb, dy_blk)


# device time: 137383 ns/iter; 1.0131x vs baseline; 1.0131x over previous
import os

import jax
import jax.numpy as jnp
from jax import lax
from jax.experimental import pallas as pl
from jax.experimental.pallas import tpu as pltpu

_KPHASES = int(os.environ.get("KPHASES", "3"))

N_RING = 8
BLK = 1024
HALF = BLK // 2
QRT = 128
LPD = HALF // QRT
M_HALF = 1024
NSLOT = 4


def _ring_pos(y, z):
    return jnp.where(y == 0, z, N_RING - 1 - z)


def _ring_to_yz(r):
    y = jnp.where(r >= N_RING // 2, 1, 0)
    z = jnp.where(r < N_RING // 2, r, N_RING - 1 - r)
    return y, z


def kernel(x, dy):
    k, m = x.shape
    _, f = dy.shape

    my_y = lax.axis_index("y")
    my_z = lax.axis_index("z")
    r = _ring_pos(my_y, my_z)
    dy_blk = lax.dynamic_slice(dy, (0, r * BLK), (k, BLK)).astype(jnp.bfloat16)

    def body(x_ref, dy_ref, out_ref, bsend, brecv, comm_cw, comm_ccw,
             out_sems, xsend_sems, xrecv_sems,
             cw_send_sems, cw_recv_sems, ccw_send_sems, ccw_recv_sems):
        my_x = lax.axis_index("x")
        my_y = lax.axis_index("y")
        my_z = lax.axis_index("z")
        r = _ring_pos(my_y, my_z)
        ry, rz = _ring_to_yz((r + 1) % N_RING)
        ly, lz = _ring_to_yz((r - 1) % N_RING)
        right = (my_x, ry, rz)
        left = (my_x, ly, lz)

        if _KPHASES >= 2:
            barrier = pltpu.get_barrier_semaphore()
            for tgt in [(1 - my_x, my_y, my_z), right, left]:
                pl.semaphore_signal(barrier, inc=1, device_id=tgt,
                                    device_id_type=pl.DeviceIdType.MESH)
            pl.semaphore_wait(barrier, 3)

        dyb = dy_ref[:, :]

        lanes = []
        for c in range(LPD):
            lanes.append(dict(cw=True, c=c, col0=c * QRT))
            lanes.append(dict(cw=False, c=c, col0=HALF + c * QRT))
        for i, ln in enumerate(lanes):
            ln["xidx"] = i
            ln["comm"] = comm_cw if ln["cw"] else comm_ccw
            ln["ss"] = cw_send_sems if ln["cw"] else ccw_send_sems
            ln["rs"] = cw_recv_sems if ln["cw"] else ccw_recv_sems
            ln["tgt"] = right if ln["cw"] else left

        other0 = (1 - my_x) * M_HALF
        b = lax.dot_general(
            x_ref[:, pl.ds(other0, M_HALF)].astype(jnp.bfloat16), dyb,
            (((0,), (0,)), ((), ())),
            preferred_element_type=jnp.float32)
        for ln in lanes:
            bsend[ln["xidx"], :, :] = (
                b[:, ln["col0"]:ln["col0"] + QRT].astype(jnp.bfloat16))

        xchg = []
        if _KPHASES >= 2:
            for ln in lanes:
                i = ln["xidx"]
                d = pltpu.make_async_remote_copy(
                    src_ref=bsend.at[i], dst_ref=brecv.at[i],
                    send_sem=xsend_sems.at[i], recv_sem=xrecv_sems.at[i],
                    device_id=(1 - my_x, my_y, my_z),
                    device_id_type=pl.DeviceIdType.MESH,
                )
                d.start()
                xchg.append(d)

        mine0 = my_x * M_HALF
        a = lax.dot_general(
            x_ref[:, pl.ds(mine0, M_HALF)].astype(jnp.bfloat16), dyb,
            (((0,), (0,)), ((), ())),
            preferred_element_type=jnp.float32)

        def ring_desc(ln, h):
            s = h % NSLOT
            rv = (h + 1) % NSLOT
            return pltpu.make_async_remote_copy(
                src_ref=ln["comm"].at[ln["c"], s],
                dst_ref=ln["comm"].at[ln["c"], rv],
                send_sem=ln["ss"].at[ln["c"], s],
                recv_sem=ln["rs"].at[ln["c"], rv],
                device_id=ln["tgt"], device_id_type=pl.DeviceIdType.MESH,
            )

        def out_copy(ln, slot, origin):
            return pltpu.make_async_copy(
                ln["comm"].at[ln["c"], slot],
                out_ref.at[:, pl.ds(origin * BLK + ln["col0"], QRT)],
                out_sems.at[ln["xidx"]])

        for ln in lanes:
            i = ln["xidx"]
            if _KPHASES >= 2:
                xchg[i].wait()
                own_c = (a[:, ln["col0"]:ln["col0"] + QRT]
                         + brecv[i, :, :].astype(jnp.float32))
            else:
                own_c = (a[:, ln["col0"]:ln["col0"] + QRT]
                         + b[:, ln["col0"]:ln["col0"] + QRT])
            ln["comm"][ln["c"], 0, :, :] = own_c.astype(jnp.bfloat16)
            if _KPHASES >= 3:
                d = ring_desc(ln, 0)
                d.start()
                ln["prev"] = d
            ln["prev_out"] = out_copy(ln, 0, r)
            ln["prev_out"].start()

        if _KPHASES < 3:
            for ln in lanes:
                ln["prev_out"].wait()
            return

        for h in range(1, N_RING - 1):
            s = h % NSLOT
            for ln in lanes:
                ln["prev"].wait_recv()
                d = ring_desc(ln, h)
                if h >= NSLOT:
                    d.wait_send()
                d.start()
                ln["prev"] = d
                o = (r - h) % N_RING if ln["cw"] else (r + h) % N_RING
                ln["prev_out"].wait()
                ln["prev_out"] = out_copy(ln, s, o)
                ln["prev_out"].start()

        s = (N_RING - 1) % NSLOT
        for ln in lanes:
            ln["prev"].wait_recv()
            o = (r + 1) % N_RING if ln["cw"] else (r - 1) % N_RING
            ln["prev_out"].wait()
            ln["prev_out"] = out_copy(ln, s, o)
            ln["prev_out"].start()

        for h in range(max(0, N_RING - 1 - NSLOT), N_RING - 1):
            for ln in lanes:
                ring_desc(ln, h).wait_send()

        for ln in lanes:
            ln["prev_out"].wait()

    out_bf = pl.pallas_call(
        body,
        out_shape=jax.ShapeDtypeStruct((M_HALF, f), jnp.bfloat16),
        in_specs=[
            pl.BlockSpec(memory_space=pltpu.VMEM),
            pl.BlockSpec(memory_space=pltpu.VMEM),
        ],
        out_specs=pl.BlockSpec(memory_space=pl.ANY),
        scratch_shapes=[
            pltpu.VMEM((2 * LPD, M_HALF, QRT), jnp.bfloat16),
            pltpu.VMEM((2 * LPD, M_HALF, QRT), jnp.bfloat16),
            pltpu.VMEM((LPD, NSLOT, M_HALF, QRT), jnp.bfloat16),
            pltpu.VMEM((LPD, NSLOT, M_HALF, QRT), jnp.bfloat16),
            pltpu.SemaphoreType.DMA((2 * LPD,)),
            pltpu.SemaphoreType.DMA((2 * LPD,)),
            pltpu.SemaphoreType.DMA((2 * LPD,)),
            pltpu.SemaphoreType.DMA((LPD, NSLOT)),
            pltpu.SemaphoreType.DMA((LPD, NSLOT)),
            pltpu.SemaphoreType.DMA((LPD, NSLOT)),
            pltpu.SemaphoreType.DMA((LPD, NSLOT)),
        ],
        compiler_params=(
            pltpu.CompilerParams(collective_id=0,
                                 vmem_limit_bytes=56 * 1024 * 1024)
            if _KPHASES >= 2
            else pltpu.CompilerParams(vmem_limit_bytes=56 * 1024 * 1024)),
    )(x, dy_blk)
    return out_bf.astype(jnp.float32)


# device time: 135777 ns/iter; 1.0250x vs baseline; 1.0118x over previous
import os

import jax
import jax.numpy as jnp
from jax import lax
from jax.experimental import pallas as pl
from jax.experimental.pallas import tpu as pltpu

_KPHASES = int(os.environ.get("KPHASES", "3"))

N_RING = 8
BLK = 1024
HALF = BLK // 2
QRT = 128
LPD = HALF // QRT
M_HALF = 1024
NSLOT = 4


def _ring_pos(y, z):
    return jnp.where(y == 0, z, N_RING - 1 - z)


def _ring_to_yz(r):
    y = jnp.where(r >= N_RING // 2, 1, 0)
    z = jnp.where(r < N_RING // 2, r, N_RING - 1 - r)
    return y, z


def kernel(x, dy):
    k, m = x.shape
    _, f = dy.shape

    my_y = lax.axis_index("y")
    my_z = lax.axis_index("z")
    r = _ring_pos(my_y, my_z)
    dy_blk = lax.dynamic_slice(dy, (0, r * BLK), (k, BLK)).astype(jnp.bfloat16)
    xb = x.astype(jnp.bfloat16)

    def body(x_ref, dy_ref, out_ref, bsend, brecv, comm_cw, comm_ccw,
             out_sems, xsend_sems, xrecv_sems,
             cw_send_sems, cw_recv_sems, ccw_send_sems, ccw_recv_sems):
        my_x = lax.axis_index("x")
        my_y = lax.axis_index("y")
        my_z = lax.axis_index("z")
        r = _ring_pos(my_y, my_z)
        ry, rz = _ring_to_yz((r + 1) % N_RING)
        ly, lz = _ring_to_yz((r - 1) % N_RING)
        right = (my_x, ry, rz)
        left = (my_x, ly, lz)

        if _KPHASES >= 2:
            barrier = pltpu.get_barrier_semaphore()
            for tgt in [(1 - my_x, my_y, my_z), right, left]:
                pl.semaphore_signal(barrier, inc=1, device_id=tgt,
                                    device_id_type=pl.DeviceIdType.MESH)
            pl.semaphore_wait(barrier, 3)

        dyb = dy_ref[:, :]

        lanes = []
        for c in range(LPD):
            lanes.append(dict(cw=True, c=c, col0=c * QRT))
            lanes.append(dict(cw=False, c=c, col0=HALF + c * QRT))
        for i, ln in enumerate(lanes):
            ln["xidx"] = i
            ln["comm"] = comm_cw if ln["cw"] else comm_ccw
            ln["ss"] = cw_send_sems if ln["cw"] else ccw_send_sems
            ln["rs"] = cw_recv_sems if ln["cw"] else ccw_recv_sems
            ln["tgt"] = right if ln["cw"] else left

        other0 = (1 - my_x) * M_HALF
        b = lax.dot_general(x_ref[:, pl.ds(other0, M_HALF)], dyb,
                            (((0,), (0,)), ((), ())),
                            preferred_element_type=jnp.float32)
        for ln in lanes:
            bsend[ln["xidx"], :, :] = (
                b[:, ln["col0"]:ln["col0"] + QRT].astype(jnp.bfloat16))

        xchg = []
        if _KPHASES >= 2:
            for ln in lanes:
                i = ln["xidx"]
                d = pltpu.make_async_remote_copy(
                    src_ref=bsend.at[i], dst_ref=brecv.at[i],
                    send_sem=xsend_sems.at[i], recv_sem=xrecv_sems.at[i],
                    device_id=(1 - my_x, my_y, my_z),
                    device_id_type=pl.DeviceIdType.MESH,
                )
                d.start()
                xchg.append(d)

        mine0 = my_x * M_HALF
        a = lax.dot_general(x_ref[:, pl.ds(mine0, M_HALF)], dyb,
                            (((0,), (0,)), ((), ())),
                            preferred_element_type=jnp.float32)

        def ring_desc(ln, h):
            s = h % NSLOT
            rv = (h + 1) % NSLOT
            return pltpu.make_async_remote_copy(
                src_ref=ln["comm"].at[ln["c"], s],
                dst_ref=ln["comm"].at[ln["c"], rv],
                send_sem=ln["ss"].at[ln["c"], s],
                recv_sem=ln["rs"].at[ln["c"], rv],
                device_id=ln["tgt"], device_id_type=pl.DeviceIdType.MESH,
            )

        def out_copy(ln, slot, origin):
            return pltpu.make_async_copy(
                ln["comm"].at[ln["c"], slot],
                out_ref.at[:, pl.ds(origin * BLK + ln["col0"], QRT)],
                out_sems.at[ln["xidx"]])

        for ln in lanes:
            i = ln["xidx"]
            if _KPHASES >= 2:
                xchg[i].wait()
                own_c = (a[:, ln["col0"]:ln["col0"] + QRT]
                         + brecv[i, :, :].astype(jnp.float32))
            else:
                own_c = (a[:, ln["col0"]:ln["col0"] + QRT]
                         + b[:, ln["col0"]:ln["col0"] + QRT])
            ln["comm"][ln["c"], 0, :, :] = own_c.astype(jnp.bfloat16)
            if _KPHASES >= 3:
                d = ring_desc(ln, 0)
                d.start()
                ln["prev"] = d
            ln["prev_out"] = out_copy(ln, 0, r)
            ln["prev_out"].start()

        if _KPHASES < 3:
            for ln in lanes:
                ln["prev_out"].wait()
            return

        for h in range(1, N_RING - 1):
            s = h % NSLOT
            for ln in lanes:
                ln["prev"].wait_recv()
                d = ring_desc(ln, h)
                if h >= NSLOT:
                    d.wait_send()
                d.start()
                ln["prev"] = d
                o = (r - h) % N_RING if ln["cw"] else (r + h) % N_RING
                ln["prev_out"].wait()
                ln["prev_out"] = out_copy(ln, s, o)
                ln["prev_out"].start()

        s = (N_RING - 1) % NSLOT
        for ln in lanes:
            ln["prev"].wait_recv()
            o = (r + 1) % N_RING if ln["cw"] else (r - 1) % N_RING
            ln["prev_out"].wait()
            ln["prev_out"] = out_copy(ln, s, o)
            ln["prev_out"].start()

        for h in range(max(0, N_RING - 1 - NSLOT), N_RING - 1):
            for ln in lanes:
                ring_desc(ln, h).wait_send()

        for ln in lanes:
            ln["prev_out"].wait()

    out_bf = pl.pallas_call(
        body,
        out_shape=jax.ShapeDtypeStruct((M_HALF, f), jnp.bfloat16),
        in_specs=[
            pl.BlockSpec(memory_space=pltpu.VMEM),
            pl.BlockSpec(memory_space=pltpu.VMEM),
        ],
        out_specs=pl.BlockSpec(memory_space=pl.ANY),
        scratch_shapes=[
            pltpu.VMEM((2 * LPD, M_HALF, QRT), jnp.bfloat16),
            pltpu.VMEM((2 * LPD, M_HALF, QRT), jnp.bfloat16),
            pltpu.VMEM((LPD, NSLOT, M_HALF, QRT), jnp.bfloat16),
            pltpu.VMEM((LPD, NSLOT, M_HALF, QRT), jnp.bfloat16),
            pltpu.SemaphoreType.DMA((2 * LPD,)),
            pltpu.SemaphoreType.DMA((2 * LPD,)),
            pltpu.SemaphoreType.DMA((2 * LPD,)),
            pltpu.SemaphoreType.DMA((LPD, NSLOT)),
            pltpu.SemaphoreType.DMA((LPD, NSLOT)),
            pltpu.SemaphoreType.DMA((LPD, NSLOT)),
            pltpu.SemaphoreType.DMA((LPD, NSLOT)),
        ],
        compiler_params=(
            pltpu.CompilerParams(collective_id=0,
                                 allow_input_fusion=[False, True])
            if _KPHASES >= 2
            else pltpu.CompilerParams(allow_input_fusion=[False, True])),
    )(xb, dy_blk)
    return out_bf.astype(jnp.float32)


# device time: 135652 ns/iter; 1.0260x vs baseline; 1.0009x over previous
import os

import jax
import jax.numpy as jnp
from jax import lax
from jax.experimental import pallas as pl
from jax.experimental.pallas import tpu as pltpu

_KPHASES = int(os.environ.get("KPHASES", "3"))

N_RING = 8
BLK = 1024
HALF = BLK // 2
QRT = 128
LPD = HALF // QRT
M_HALF = 1024
NSLOT = 4


def _ring_pos(y, z):
    return jnp.where(y == 0, z, N_RING - 1 - z)


def _ring_to_yz(r):
    y = jnp.where(r >= N_RING // 2, 1, 0)
    z = jnp.where(r < N_RING // 2, r, N_RING - 1 - r)
    return y, z


def kernel(x, dy):
    k, m = x.shape
    _, f = dy.shape

    my_y = lax.axis_index("y")
    my_z = lax.axis_index("z")
    r = _ring_pos(my_y, my_z)
    dy_blk = lax.dynamic_slice(dy, (0, r * BLK), (k, BLK)).astype(jnp.bfloat16)
    xb = x.astype(jnp.bfloat16)

    def body(x_ref, dy_ref, out_ref, bsend, brecv, comm_cw, comm_ccw,
             out_sems, xsend_sems, xrecv_sems,
             cw_send_sems, cw_recv_sems, ccw_send_sems, ccw_recv_sems):
        my_x = lax.axis_index("x")
        my_y = lax.axis_index("y")
        my_z = lax.axis_index("z")
        r = _ring_pos(my_y, my_z)
        ry, rz = _ring_to_yz((r + 1) % N_RING)
        ly, lz = _ring_to_yz((r - 1) % N_RING)
        right = (my_x, ry, rz)
        left = (my_x, ly, lz)

        if _KPHASES >= 2:
            barrier = pltpu.get_barrier_semaphore()
            for tgt in [(1 - my_x, my_y, my_z), right, left]:
                pl.semaphore_signal(barrier, inc=1, device_id=tgt,
                                    device_id_type=pl.DeviceIdType.MESH)
            pl.semaphore_wait(barrier, 3)

        dyb = dy_ref[:, :]

        lanes = []
        for c in range(LPD):
            lanes.append(dict(cw=True, c=c, col0=c * QRT))
            lanes.append(dict(cw=False, c=c, col0=HALF + c * QRT))
        for i, ln in enumerate(lanes):
            ln["xidx"] = i
            ln["comm"] = comm_cw if ln["cw"] else comm_ccw
            ln["ss"] = cw_send_sems if ln["cw"] else ccw_send_sems
            ln["rs"] = cw_recv_sems if ln["cw"] else ccw_recv_sems
            ln["tgt"] = right if ln["cw"] else left

        other0 = (1 - my_x) * M_HALF
        b = lax.dot_general(x_ref[:, pl.ds(other0, M_HALF)], dyb,
                            (((0,), (0,)), ((), ())),
                            preferred_element_type=jnp.float32)
        for ln in lanes:
            bsend[ln["xidx"], :, :] = (
                b[:, ln["col0"]:ln["col0"] + QRT].astype(jnp.bfloat16))

        xchg = []
        if _KPHASES >= 2:
            for ln in lanes:
                i = ln["xidx"]
                d = pltpu.make_async_remote_copy(
                    src_ref=bsend.at[i], dst_ref=brecv.at[i],
                    send_sem=xsend_sems.at[i], recv_sem=xrecv_sems.at[i],
                    device_id=(1 - my_x, my_y, my_z),
                    device_id_type=pl.DeviceIdType.MESH,
                )
                d.start()
                xchg.append(d)

        mine0 = my_x * M_HALF
        a = lax.dot_general(x_ref[:, pl.ds(mine0, M_HALF)], dyb,
                            (((0,), (0,)), ((), ())),
                            preferred_element_type=jnp.float32)

        def ring_desc(ln, h):
            s = h % NSLOT
            rv = (h + 1) % NSLOT
            return pltpu.make_async_remote_copy(
                src_ref=ln["comm"].at[ln["c"], s],
                dst_ref=ln["comm"].at[ln["c"], rv],
                send_sem=ln["ss"].at[ln["c"], s],
                recv_sem=ln["rs"].at[ln["c"], rv],
                device_id=ln["tgt"], device_id_type=pl.DeviceIdType.MESH,
            )

        def out_copy(ln, slot, origin):
            return pltpu.make_async_copy(
                ln["comm"].at[ln["c"], slot],
                out_ref.at[:, pl.ds(origin * BLK + ln["col0"], QRT)],
                out_sems.at[ln["xidx"]])

        for ln in lanes:
            i = ln["xidx"]
            if _KPHASES >= 2:
                xchg[i].wait()
                own_c = (a[:, ln["col0"]:ln["col0"] + QRT]
                         + brecv[i, :, :].astype(jnp.float32))
            else:
                own_c = (a[:, ln["col0"]:ln["col0"] + QRT]
                         + b[:, ln["col0"]:ln["col0"] + QRT])
            ln["comm"][ln["c"], 0, :, :] = own_c.astype(jnp.bfloat16)
            if _KPHASES >= 3:
                d = ring_desc(ln, 0)
                d.start()
                ln["prev"] = d
            ln["prev_out"] = out_copy(ln, 0, r)
            ln["prev_out"].start()

        if _KPHASES < 3:
            for ln in lanes:
                ln["prev_out"].wait()
            return

        for h in range(1, N_RING - 1):
            s = h % NSLOT
            for ln in lanes:
                ln["prev"].wait_recv()
                d = ring_desc(ln, h)
                if h >= NSLOT:
                    d.wait_send()
                d.start()
                ln["prev"] = d
                o = (r - h) % N_RING if ln["cw"] else (r + h) % N_RING
                ln["prev_out"].wait()
                ln["prev_out"] = out_copy(ln, s, o)
                ln["prev_out"].start()

        s = (N_RING - 1) % NSLOT
        for ln in lanes:
            ln["prev"].wait_recv()
            o = (r + 1) % N_RING if ln["cw"] else (r - 1) % N_RING
            ln["prev_out"].wait()
            ln["prev_out"] = out_copy(ln, s, o)
            ln["prev_out"].start()

        for h in range(max(0, N_RING - 1 - NSLOT), N_RING - 1):
            for ln in lanes:
                ring_desc(ln, h).wait_send()

        for ln in lanes:
            ln["prev_out"].wait()

    out_bf = pl.pallas_call(
        body,
        out_shape=jax.ShapeDtypeStruct((M_HALF, f), jnp.bfloat16),
        in_specs=[
            pl.BlockSpec(memory_space=pltpu.VMEM),
            pl.BlockSpec(memory_space=pltpu.VMEM),
        ],
        out_specs=pl.BlockSpec(memory_space=pl.ANY),
        scratch_shapes=[
            pltpu.VMEM((2 * LPD, M_HALF, QRT), jnp.bfloat16),
            pltpu.VMEM((2 * LPD, M_HALF, QRT), jnp.bfloat16),
            pltpu.VMEM((LPD, NSLOT, M_HALF, QRT), jnp.bfloat16),
            pltpu.VMEM((LPD, NSLOT, M_HALF, QRT), jnp.bfloat16),
            pltpu.SemaphoreType.DMA((2 * LPD,)),
            pltpu.SemaphoreType.DMA((2 * LPD,)),
            pltpu.SemaphoreType.DMA((2 * LPD,)),
            pltpu.SemaphoreType.DMA((LPD, NSLOT)),
            pltpu.SemaphoreType.DMA((LPD, NSLOT)),
            pltpu.SemaphoreType.DMA((LPD, NSLOT)),
            pltpu.SemaphoreType.DMA((LPD, NSLOT)),
        ],
        compiler_params=(pltpu.CompilerParams(collective_id=0)
                         if _KPHASES >= 2 else pltpu.CompilerParams()),
    )(xb, dy_blk)
    return out_bf.astype(jnp.float32)
